# baseline (speedup 1.0000x reference)
import functools

import jax
import jax.numpy as jnp
from jax import lax
from jax.experimental import pallas as pl
from jax.experimental.pallas import tpu as pltpu
from jax.experimental.pallas import tpu_sc as plsc

B, F, L, D = 1024, 26, 50, 32
G = B * F
NW = 32
GPW = G // NW
CG = 2
IPC = CG * L
NCHUNK = GPW // CG
NBUF = 4
BPW = B // NW
CPB = F // CG

_mesh = plsc.VectorSubcoreMesh(core_axis_name="c", subcore_axis_name="s")


@functools.partial(
    pl.kernel,
    out_type=jax.ShapeDtypeStruct((B, F, D, L), jnp.float32),
    mesh=_mesh,
    scratch_types=[
        pltpu.VMEM((NCHUNK, IPC), jnp.int32),
        pltpu.VMEM((NBUF, IPC, D), jnp.float32),
        pltpu.VMEM((NBUF, CG, D, L), jnp.float32),
    ]
    + [pltpu.SemaphoreType.DMA] * (3 * NBUF),
    compiler_params=pltpu.CompilerParams(
        needs_layout_passes=False, use_tc_tiling_on_sc=False
    ),
)
def _emb_lookup(idx_hbm, table_hbm, out_hbm, idx_v, rows_v, out_v, *sems):
    sem_g = sems[:NBUF]
    sem_w = sems[NBUF : 2 * NBUF]
    sem_h = sems[2 * NBUF :]
    sid = lax.axis_index("s")
    wid = sid * 2 + lax.axis_index("c")
    iota16 = lax.iota(jnp.int32, 16)

    def out_slice(c):
        b0 = wid * BPW + c // CPB
        f0 = (c % CPB) * CG
        return out_hbm.at[b0, pl.ds(f0, CG)]

    pltpu.sync_copy(idx_hbm.at[pl.ds(wid * NCHUNK, NCHUNK)], idx_v)

    H0 = 48

    def fire_gather(c, b):
        row = idx_v.at[c]
        rv = rows_v.at[b]
        pltpu.async_copy(
            table_hbm.at[row.at[pl.ds(0, H0)]], rv.at[pl.ds(0, H0)], sem_g[b]
        )
        pltpu.async_copy(
            table_hbm.at[row.at[pl.ds(H0, IPC - H0)]],
            rv.at[pl.ds(H0, IPC - H0)],
            sem_h[b],
        )

    for b in range(NBUF):
        fire_gather(b, b)

    def ring_body(p, _):
        for b in range(NBUF):
            c = p * NBUF + b
            rv = rows_v.at[b]
            ov = out_v.at[b]

            pltpu.make_async_copy(
                table_hbm.at[idx_v.at[c].at[pl.ds(0, H0)]],
                rv.at[pl.ds(0, H0)],
                sem_g[b],
            ).wait()
            pltpu.make_async_copy(
                table_hbm.at[idx_v.at[c].at[pl.ds(H0, IPC - H0)]],
                rv.at[pl.ds(H0, IPC - H0)],
                sem_h[b],
            ).wait()

            @pl.when(p > 0)
            def _():
                pltpu.make_async_copy(ov, out_slice(c - NBUF), sem_w[b]).wait()

            def row_body(l, _):
                ol = jnp.full((16,), l, dtype=jnp.int32)
                for g in range(CG):
                    r = g * L + l
                    og = jnp.full((16,), g, dtype=jnp.int32)
                    v0 = rv[r, pl.ds(0, 16)]
                    v1 = rv[r, pl.ds(16, 16)]
                    plsc.store_scatter(ov, [og, iota16, ol], v0)
                    plsc.store_scatter(ov, [og, iota16 + 16, ol], v1)
                return 0

            lax.fori_loop(0, L, row_body, 0, unroll=2)

            @pl.when(c + NBUF < NCHUNK)
            def _():
                fire_gather(c + NBUF, b)

            pltpu.async_copy(ov, out_slice(c), sem_w[b])
        return 0

    lax.fori_loop(0, NCHUNK // NBUF, ring_body, 0)

    for b in range(NBUF):
        c = NCHUNK - NBUF + b
        pltpu.make_async_copy(out_v.at[b], out_slice(c), sem_w[b]).wait()


def kernel(inputs, table):
    idx = inputs.reshape(G // CG, IPC).astype(jnp.int32)
    return _emb_lookup(idx, table)


# speedup vs baseline: 1.0017x; 1.0017x over previous
import functools

import jax
import jax.numpy as jnp
from jax import lax
from jax.experimental import pallas as pl
from jax.experimental.pallas import tpu as pltpu
from jax.experimental.pallas import tpu_sc as plsc

B, F, L, D = 1024, 26, 50, 32
G = B * F
NW = 32
GPW = G // NW
CG = 2
IPC = CG * L
NCHUNK = GPW // CG
NBUF = 4
BPW = B // NW
CPB = F // CG

_mesh = plsc.VectorSubcoreMesh(core_axis_name="c", subcore_axis_name="s")


@functools.partial(
    pl.kernel,
    out_type=jax.ShapeDtypeStruct((B, F, D, L), jnp.float32),
    mesh=_mesh,
    scratch_types=[
        pltpu.VMEM((NCHUNK, IPC), jnp.int32),
        pltpu.VMEM((NBUF, IPC, D), jnp.float32),
        pltpu.VMEM((NBUF, CG, D, L), jnp.float32),
    ]
    + [pltpu.SemaphoreType.DMA] * (2 * NBUF),
    compiler_params=pltpu.CompilerParams(
        needs_layout_passes=False, use_tc_tiling_on_sc=False
    ),
)
def _emb_lookup(idx_hbm, table_hbm, out_hbm, idx_v, rows_v, out_v, *sems):
    sem_g = sems[:NBUF]
    sem_w = sems[NBUF:]
    sid = lax.axis_index("s")
    wid = sid * 2 + lax.axis_index("c")
    iota16 = lax.iota(jnp.int32, 16)

    def out_slice(c):
        b0 = wid * BPW + c // CPB
        f0 = (c % CPB) * CG
        return out_hbm.at[b0, pl.ds(f0, CG)]

    pltpu.sync_copy(idx_hbm.at[pl.ds(wid * NCHUNK, NCHUNK)], idx_v)

    def fire_gather(c, b):
        pltpu.async_copy(table_hbm.at[idx_v.at[c]], rows_v.at[b], sem_g[b])

    for b in range(NBUF):
        fire_gather(b, b)

    def ring_body(p, _):
        for b in range(NBUF):
            c = p * NBUF + b
            rv = rows_v.at[b]
            ov = out_v.at[b]

            pltpu.make_async_copy(table_hbm.at[idx_v.at[c]], rv, sem_g[b]).wait()

            @pl.when(p > 0)
            def _():
                pltpu.make_async_copy(ov, out_slice(c - NBUF), sem_w[b]).wait()

            def row_body(l, _):
                ol = jnp.full((16,), l, dtype=jnp.int32)
                for g in range(CG):
                    r = g * L + l
                    og = jnp.full((16,), g, dtype=jnp.int32)
                    v0 = rv[r, pl.ds(0, 16)]
                    v1 = rv[r, pl.ds(16, 16)]
                    plsc.store_scatter(ov, [og, iota16, ol], v0)
                    plsc.store_scatter(ov, [og, iota16 + 16, ol], v1)
                return 0

            lax.fori_loop(0, L, row_body, 0, unroll=2)

            @pl.when(c + NBUF < NCHUNK)
            def _():
                fire_gather(c + NBUF, b)

            pltpu.async_copy(ov, out_slice(c), sem_w[b])
        return 0

    lax.fori_loop(0, NCHUNK // NBUF, ring_body, 0)

    for b in range(NBUF):
        c = NCHUNK - NBUF + b
        pltpu.make_async_copy(out_v.at[b], out_slice(c), sem_w[b]).wait()


def kernel(inputs, table):
    idx = inputs.reshape(G // CG, IPC).astype(jnp.int32)
    return _emb_lookup(idx, table)
